# Initial kernel scaffold; baseline (speedup 1.0000x reference)
import jax
import jax.numpy as jnp
from jax.experimental import pallas as pl


def kernel(x, edge_index, W_e1, b_e1, W_e2, b_e2, W_d1, b_d1, W_d2, b_d2):
    raise NotImplementedError("write your pallas kernel here")


# speedup vs baseline: 9.7146x; 9.7146x over previous
import functools

import jax
import jax.numpy as jnp
from jax import lax
from jax.experimental import pallas as pl
from jax.experimental.pallas import tpu as pltpu
from jax.experimental.pallas import tpu_sc as plsc

_N = 10000
_E = 160000
_TILES = 16
_WB = 632
_LAST = _N - (_TILES - 1) * _WB
_CH = 128

_R = 2000
_G = _N // _R

_mesh = plsc.VectorSubcoreMesh(core_axis_name="c", subcore_axis_name="s")


def _zero_table(z_ref, table, t):
    start = pl.multiple_of(t * _WB, 8)

    @pl.when(t < _TILES - 1)
    def _():
        pltpu.sync_copy(z_ref, table.at[pl.ds(start, _WB)])

    @pl.when(t == _TILES - 1)
    def _():
        pltpu.sync_copy(z_ref.at[pl.ds(0, _LAST)], table.at[pl.ds(start, _LAST)])


def _write_table(table, out_ref, c, t):
    start = pl.multiple_of(t * _WB, 8)

    @pl.when(t < _TILES - 1)
    def _():
        pltpu.sync_copy(table.at[pl.ds(start, _WB)], out_ref.at[c, pl.ds(start, _WB)])

    @pl.when(t == _TILES - 1)
    def _():
        pltpu.sync_copy(table.at[pl.ds(start, _LAST)], out_ref.at[c, pl.ds(start, _LAST)])


def _sc_deg():
    n_chunks = (_E // 2) // _CH
    iters = (n_chunks + _TILES - 1) // _TILES

    @functools.partial(
        pl.kernel,
        out_type=jax.ShapeDtypeStruct((2, _N, 128), jnp.float32),
        mesh=_mesh,
        scratch_types=[
            pltpu.VMEM((_CH,), jnp.int32),
            pltpu.VMEM((_CH, 128), jnp.float32),
            pltpu.VMEM_SHARED((_N, 128), jnp.float32),
        ],
    )
    def k(dst_ref, oh_ref, z_ref, out_ref, didx, ones_v, table):
        c = lax.axis_index("c")
        t = lax.axis_index("s")
        pltpu.sync_copy(oh_ref, ones_v)
        _zero_table(z_ref, table, t)
        plsc.subcore_barrier()

        def body(j, carry):
            chunk = j * _TILES + t

            @pl.when(chunk < n_chunks)
            def _():
                base = pl.multiple_of(c * (_E // 2) + chunk * _CH, _CH)
                pltpu.sync_copy(dst_ref.at[pl.ds(base, _CH)], didx)
                pltpu.sync_copy(ones_v, table.at[didx], add=True)

            return carry

        lax.fori_loop(0, iters, body, 0)
        plsc.subcore_barrier()
        _write_table(table, out_ref, c, t)

    return k


def _sc_agg(d, cat):
    epc = _E if cat else _E // 2
    n_chunks = epc // _CH
    iters = (n_chunks + _TILES - 1) // _TILES

    @functools.partial(
        pl.kernel,
        out_type=jax.ShapeDtypeStruct((2, _N, d), jnp.float32),
        mesh=_mesh,
        scratch_types=[
            pltpu.VMEM((_CH,), jnp.int32),
            pltpu.VMEM((_CH,), jnp.int32),
            pltpu.VMEM((_CH, d), jnp.float32),
            pltpu.VMEM_SHARED((_N, d), jnp.float32),
            pltpu.SemaphoreType.DMA,
        ],
    )
    def k(src_ref, dst_ref, s_ref, z_ref, out_ref, sidx, didx, rows, table, sem):
        c = lax.axis_index("c")
        t = lax.axis_index("s")
        _zero_table(z_ref, table, t)
        plsc.subcore_barrier()

        def body(j, carry):
            chunk = j * _TILES + t

            @pl.when(chunk < n_chunks)
            def _():
                if cat:
                    base = pl.multiple_of(chunk * _CH, _CH)
                    pltpu.sync_copy(src_ref.at[c, pl.ds(base, _CH)], sidx)
                else:
                    base = pl.multiple_of(c * epc + chunk * _CH, _CH)
                    pltpu.sync_copy(src_ref.at[pl.ds(base, _CH)], sidx)
                pltpu.sync_copy(dst_ref.at[pl.ds(base, _CH)], didx)
                pltpu.async_copy(s_ref.at[sidx], rows, sem).wait()
                pltpu.sync_copy(rows, table.at[didx], add=True)

            return carry

        lax.fori_loop(0, iters, body, 0)
        plsc.subcore_barrier()
        _write_table(table, out_ref, c, t)

    return k


def _tc_pre1(x, W, degtab):
    din, dout = W.shape

    def body(x_ref, w_ref, dg_ref, dinv_ref, s_ref):
        dinv = lax.rsqrt(1.0 + dg_ref[0, :, 0:1] + dg_ref[1, :, 0:1])
        h = jnp.dot(x_ref[...], w_ref[...], preferred_element_type=jnp.float32)
        dinv_ref[...] = dinv
        s_ref[...] = dinv * h

    return pl.pallas_call(
        body,
        grid=(_G,),
        in_specs=[
            pl.BlockSpec((_R, din), lambda i: (i, 0)),
            pl.BlockSpec((din, dout), lambda i: (0, 0)),
            pl.BlockSpec((2, _R, 128), lambda i: (0, i, 0)),
        ],
        out_specs=[
            pl.BlockSpec((_R, 1), lambda i: (i, 0)),
            pl.BlockSpec((_R, dout), lambda i: (i, 0)),
        ],
        out_shape=[
            jax.ShapeDtypeStruct((_N, 1), jnp.float32),
            jax.ShapeDtypeStruct((_N, dout), jnp.float32),
        ],
    )(x, W, degtab)


def _tc_mid(p, s_prev, dinv, b, W, mode, stack_out=False, take=None, pad_to=None):
    din, dout = W.shape
    sin = s_prev.shape[1]

    def body(p_ref, s_ref, dv_ref, b_ref, w_ref, o_ref):
        dv = dv_ref[...]
        agg = p_ref[0] + p_ref[1] + s_ref[...]
        if take is not None:
            agg = agg[:, :take]
        h = dv * agg + b_ref[...]
        if mode == "relu":
            h = jnp.maximum(h, 0.0)
        elif mode == "norm":
            n = jnp.sqrt(jnp.sum(h * h, axis=1, keepdims=True))
            h = h / jnp.maximum(n, 1e-12)
        s_next = dv * jnp.dot(h, w_ref[...], preferred_element_type=jnp.float32)
        if stack_out:
            o_ref[0] = s_next[:, : dout // 2]
            o_ref[1] = s_next[:, dout // 2 :]
        elif pad_to is not None:
            o_ref[...] = jnp.concatenate(
                [s_next, jnp.zeros((_R, pad_to - dout), jnp.float32)], axis=1
            )
        else:
            o_ref[...] = s_next

    if stack_out:
        out_spec = pl.BlockSpec((2, _R, dout // 2), lambda i: (0, i, 0))
        out_shape = jax.ShapeDtypeStruct((2, _N, dout // 2), jnp.float32)
    else:
        ocols = pad_to if pad_to is not None else dout
        out_spec = pl.BlockSpec((_R, ocols), lambda i: (i, 0))
        out_shape = jax.ShapeDtypeStruct((_N, ocols), jnp.float32)

    return pl.pallas_call(
        body,
        grid=(_G,),
        in_specs=[
            pl.BlockSpec((2, _R, sin), lambda i: (0, i, 0)),
            pl.BlockSpec((_R, sin), lambda i: (i, 0)),
            pl.BlockSpec((_R, 1), lambda i: (i, 0)),
            pl.BlockSpec((1, din), lambda i: (0, 0)),
            pl.BlockSpec((din, dout), lambda i: (0, 0)),
        ],
        out_specs=out_spec,
        out_shape=out_shape,
    )(p, s_prev, dinv, b, W)


def _tc_post(p4, s4stk, dinv, b):
    half = p4.shape[2]

    def body(p_ref, s_ref, dv_ref, b_ref, o_ref):
        dv = dv_ref[...]
        left = dv * (p_ref[0] + s_ref[0])
        right = dv * (p_ref[1] + s_ref[1])
        o_ref[...] = jnp.concatenate([left, right], axis=1) + b_ref[...]

    return pl.pallas_call(
        body,
        grid=(_G,),
        in_specs=[
            pl.BlockSpec((2, _R, half), lambda i: (0, i, 0)),
            pl.BlockSpec((2, _R, half), lambda i: (0, i, 0)),
            pl.BlockSpec((_R, 1), lambda i: (i, 0)),
            pl.BlockSpec((1, 2 * half), lambda i: (0, 0)),
        ],
        out_specs=pl.BlockSpec((_R, 2 * half), lambda i: (i, 0)),
        out_shape=jax.ShapeDtypeStruct((_N, 2 * half), jnp.float32),
    )(p4, s4stk, dinv, b)


def kernel(x, edge_index, W_e1, b_e1, W_e2, b_e2, W_d1, b_d1, W_d2, b_d2):
    src = edge_index[0]
    dst = edge_index[1]
    src2 = jnp.stack([src, src + _N])
    onehot = jnp.zeros((_CH, 128), jnp.float32).at[:, 0].set(1.0)

    z128 = jnp.zeros((_WB, 128), jnp.float32)

    degtab = _sc_deg()(dst, onehot, z128)
    dinv, s1 = _tc_pre1(x, W_e1, degtab)
    p1 = _sc_agg(128, cat=False)(src, dst, s1, z128)
    s2 = _tc_mid(p1, s1, dinv, b_e1.reshape(1, -1), W_e2, "relu", pad_to=128)
    p2 = _sc_agg(128, cat=False)(src, dst, s2, z128)
    s3 = _tc_mid(p2, s2, dinv, b_e2.reshape(1, -1), W_d1, "norm", take=64)
    p3 = _sc_agg(128, cat=False)(src, dst, s3, z128)
    s4stk = _tc_mid(p3, s3, dinv, b_d1.reshape(1, -1), W_d2, "relu", stack_out=True)
    p4 = _sc_agg(128, cat=True)(src2, dst, s4stk.reshape(2 * _N, 128), z128)
    x_hat = _tc_post(p4, s4stk, dinv, b_d2.reshape(1, -1))
    return x_hat


# speedup vs baseline: 11.4145x; 1.1750x over previous
import functools

import jax
import jax.numpy as jnp
from jax import lax
from jax.experimental import pallas as pl
from jax.experimental.pallas import tpu as pltpu
from jax.experimental.pallas import tpu_sc as plsc

_N = 10000
_E = 160000
_TILES = 16
_WB = 632
_LAST = _N - (_TILES - 1) * _WB
_CH = 128

_R = 2000
_G = _N // _R

_mesh = plsc.VectorSubcoreMesh(core_axis_name="c", subcore_axis_name="s")


def _zero_table(z_ref, table, t):
    start = pl.multiple_of(t * _WB, 8)

    @pl.when(t < _TILES - 1)
    def _():
        pltpu.sync_copy(z_ref, table.at[pl.ds(start, _WB)])

    @pl.when(t == _TILES - 1)
    def _():
        pltpu.sync_copy(z_ref.at[pl.ds(0, _LAST)], table.at[pl.ds(start, _LAST)])


def _write_table(table, out_ref, c, t):
    start = pl.multiple_of(t * _WB, 8)

    @pl.when(t < _TILES - 1)
    def _():
        pltpu.sync_copy(table.at[pl.ds(start, _WB)], out_ref.at[c, pl.ds(start, _WB)])

    @pl.when(t == _TILES - 1)
    def _():
        pltpu.sync_copy(table.at[pl.ds(start, _LAST)], out_ref.at[c, pl.ds(start, _LAST)])


def _sc_deg():
    n_chunks = (_E // 2) // _CH
    iters = (n_chunks + _TILES - 1) // _TILES

    @functools.partial(
        pl.kernel,
        out_type=jax.ShapeDtypeStruct((2, _N, 128), jnp.float32),
        mesh=_mesh,
        scratch_types=[
            pltpu.VMEM((_CH,), jnp.int32),
            pltpu.VMEM((_CH, 128), jnp.float32),
            pltpu.VMEM_SHARED((_N, 128), jnp.float32),
        ],
    )
    def k(dst_ref, oh_ref, z_ref, out_ref, didx, ones_v, table):
        c = lax.axis_index("c")
        t = lax.axis_index("s")
        pltpu.sync_copy(oh_ref, ones_v)
        _zero_table(z_ref, table, t)
        plsc.subcore_barrier()

        def body(j, carry):
            chunk = j * _TILES + t

            @pl.when(chunk < n_chunks)
            def _():
                base = pl.multiple_of(c * (_E // 2) + chunk * _CH, _CH)
                pltpu.sync_copy(dst_ref.at[pl.ds(base, _CH)], didx)
                pltpu.sync_copy(ones_v, table.at[didx], add=True)

            return carry

        lax.fori_loop(0, iters, body, 0)
        plsc.subcore_barrier()
        _write_table(table, out_ref, c, t)

    return k


def _sc_agg(d):
    epc = _E // 2
    n_chunks = epc // _CH
    iters = (n_chunks + _TILES - 1) // _TILES

    @functools.partial(
        pl.kernel,
        out_type=jax.ShapeDtypeStruct((2, _N, d), jnp.float32),
        mesh=_mesh,
        scratch_types=[
            pltpu.VMEM((_CH,), jnp.int32),
            pltpu.VMEM((_CH,), jnp.int32),
            pltpu.VMEM((_CH, d), jnp.float32),
            pltpu.VMEM_SHARED((_N, d), jnp.float32),
            pltpu.SemaphoreType.DMA,
        ],
    )
    def k(src_ref, dst_ref, s_ref, z_ref, out_ref, sidx, didx, rows, table, sem):
        c = lax.axis_index("c")
        t = lax.axis_index("s")
        _zero_table(z_ref, table, t)
        plsc.subcore_barrier()

        def body(j, carry):
            chunk = j * _TILES + t

            @pl.when(chunk < n_chunks)
            def _():
                base = pl.multiple_of(c * epc + chunk * _CH, _CH)
                pltpu.sync_copy(src_ref.at[pl.ds(base, _CH)], sidx)
                pltpu.sync_copy(dst_ref.at[pl.ds(base, _CH)], didx)
                pltpu.async_copy(s_ref.at[sidx], rows, sem).wait()
                pltpu.sync_copy(rows, table.at[didx], add=True)

            return carry

        lax.fori_loop(0, iters, body, 0)
        plsc.subcore_barrier()
        _write_table(table, out_ref, c, t)

    return k


def _tc_pre1(x, W, degtab):
    din, dout = W.shape

    def body(x_ref, w_ref, dg_ref, dinv_ref, s_ref):
        dinv = lax.rsqrt(1.0 + dg_ref[0, :, 0:1] + dg_ref[1, :, 0:1])
        h = jnp.dot(x_ref[...], w_ref[...], preferred_element_type=jnp.float32)
        dinv_ref[...] = dinv
        s_ref[...] = dinv * h

    return pl.pallas_call(
        body,
        grid=(_G,),
        in_specs=[
            pl.BlockSpec((_R, din), lambda i: (i, 0)),
            pl.BlockSpec((din, dout), lambda i: (0, 0)),
            pl.BlockSpec((2, _R, 128), lambda i: (0, i, 0)),
        ],
        out_specs=[
            pl.BlockSpec((_R, 1), lambda i: (i, 0)),
            pl.BlockSpec((_R, dout), lambda i: (i, 0)),
        ],
        out_shape=[
            jax.ShapeDtypeStruct((_N, 1), jnp.float32),
            jax.ShapeDtypeStruct((_N, dout), jnp.float32),
        ],
    )(x, W, degtab)


def _tc_mid(p, s_prev, dinv, b, W, mode, stack_out=False, take=None, pad_to=None):
    if W is None:
        din = dout = b.shape[1]
    else:
        din, dout = W.shape
    sin = s_prev.shape[1]

    def body(*refs):
        if W is None:
            p_ref, s_ref, dv_ref, b_ref, o_ref = refs
        else:
            p_ref, s_ref, dv_ref, b_ref, w_ref, o_ref = refs
        dv = dv_ref[...]
        agg = p_ref[0] + p_ref[1] + s_ref[...]
        if take is not None:
            agg = agg[:, :take]
        h = dv * agg + b_ref[...]
        if mode == "relu":
            h = jnp.maximum(h, 0.0)
        elif mode == "norm":
            n = jnp.sqrt(jnp.sum(h * h, axis=1, keepdims=True))
            h = h / jnp.maximum(n, 1e-12)
        if W is None:
            s_next = dv * h
        else:
            s_next = dv * jnp.dot(h, w_ref[...], preferred_element_type=jnp.float32)
        if stack_out:
            o_ref[0] = s_next[:, : dout // 2]
            o_ref[1] = s_next[:, dout // 2 :]
        elif pad_to is not None:
            o_ref[...] = jnp.concatenate(
                [s_next, jnp.zeros((_R, pad_to - dout), jnp.float32)], axis=1
            )
        else:
            o_ref[...] = s_next

    if stack_out:
        out_spec = pl.BlockSpec((2, _R, dout // 2), lambda i: (0, i, 0))
        out_shape = jax.ShapeDtypeStruct((2, _N, dout // 2), jnp.float32)
    else:
        ocols = pad_to if pad_to is not None else dout
        out_spec = pl.BlockSpec((_R, ocols), lambda i: (i, 0))
        out_shape = jax.ShapeDtypeStruct((_N, ocols), jnp.float32)

    in_specs = [
        pl.BlockSpec((2, _R, sin), lambda i: (0, i, 0)),
        pl.BlockSpec((_R, sin), lambda i: (i, 0)),
        pl.BlockSpec((_R, 1), lambda i: (i, 0)),
        pl.BlockSpec((1, din), lambda i: (0, 0)),
    ]
    args = [p, s_prev, dinv, b]
    if W is not None:
        in_specs.append(pl.BlockSpec((din, dout), lambda i: (0, 0)))
        args.append(W)

    return pl.pallas_call(
        body,
        grid=(_G,),
        in_specs=in_specs,
        out_specs=out_spec,
        out_shape=out_shape,
    )(*args)


def _tc_post(p4, h3t, dinv, b, W):
    din, dout = W.shape

    def body(p_ref, s_ref, dv_ref, b_ref, w_ref, o_ref):
        tot = p_ref[0] + p_ref[1] + s_ref[...]
        mm = jnp.dot(tot, w_ref[...], preferred_element_type=jnp.float32)
        o_ref[...] = dv_ref[...] * mm + b_ref[...]

    return pl.pallas_call(
        body,
        grid=(_G,),
        in_specs=[
            pl.BlockSpec((2, _R, din), lambda i: (0, i, 0)),
            pl.BlockSpec((_R, din), lambda i: (i, 0)),
            pl.BlockSpec((_R, 1), lambda i: (i, 0)),
            pl.BlockSpec((1, dout), lambda i: (0, 0)),
            pl.BlockSpec((din, dout), lambda i: (0, 0)),
        ],
        out_specs=pl.BlockSpec((_R, dout), lambda i: (i, 0)),
        out_shape=jax.ShapeDtypeStruct((_N, dout), jnp.float32),
    )(p4, h3t, dinv, b, W)


def kernel(x, edge_index, W_e1, b_e1, W_e2, b_e2, W_d1, b_d1, W_d2, b_d2):
    src = edge_index[0]
    dst = edge_index[1]
    onehot = jnp.zeros((_CH, 128), jnp.float32).at[:, 0].set(1.0)
    z128 = jnp.zeros((_WB, 128), jnp.float32)

    degtab = _sc_deg()(dst, onehot, z128)
    dinv, s1 = _tc_pre1(x, W_e1, degtab)
    p1 = _sc_agg(128)(src, dst, s1, z128)
    s2 = _tc_mid(p1, s1, dinv, b_e1.reshape(1, -1), W_e2, "relu", pad_to=128)
    p2 = _sc_agg(128)(src, dst, s2, z128)
    s3 = _tc_mid(p2, s2, dinv, b_e2.reshape(1, -1), W_d1, "norm", take=64)
    p3 = _sc_agg(128)(src, dst, s3, z128)
    h3t = _tc_mid(p3, s3, dinv, b_d1.reshape(1, -1), None, "relu")
    p4 = _sc_agg(128)(src, dst, h3t, z128)
    x_hat = _tc_post(p4, h3t, dinv, b_d2.reshape(1, -1), W_d2)
    return x_hat
